# baseline (speedup 1.0000x reference)
import functools

import jax
import jax.numpy as jnp
from jax import lax
from jax.experimental import pallas as pl
from jax.experimental.pallas import tpu as pltpu
from jax.experimental.pallas import tpu_sc as plsc

VOCAB = 1000000
MAX_LEN = 2048
EMB = 64
B, L = 1024, 200
N_IDX = B * L
NC, NS = 2, 16
NW = NC * NS
CHUNK = 128
BC = B // CHUNK
LPW = L // (NW // BC)
NBUF = 2

_mesh = plsc.VectorSubcoreMesh(
    core_axis_name="c", subcore_axis_name="s", num_cores=NC, num_subcores=NS
)


@functools.partial(
    pl.kernel,
    out_type=jax.ShapeDtypeStruct((L, 8, BC, 8, CHUNK), jnp.float32),
    mesh=_mesh,
    compiler_params=pltpu.CompilerParams(use_tc_tiling_on_sc=False,
                                         needs_layout_passes=False),
    scratch_types=[
        pltpu.VMEM((NBUF, CHUNK), jnp.int32),
        pltpu.VMEM((NBUF, CHUNK), jnp.int32),
        pltpu.VMEM((CHUNK,), jnp.int32),
        pltpu.VMEM((NBUF, CHUNK, EMB), jnp.float32),
        pltpu.VMEM((NBUF, 8, 1, 8, CHUNK), jnp.float32),
        pltpu.VMEM_SHARED((MAX_LEN, EMB), jnp.float32),
        pltpu.SemaphoreType.DMA,
        pltpu.SemaphoreType.DMA,
        pltpu.SemaphoreType.DMA,
        pltpu.SemaphoreType.DMA,
    ],
)
def _lookup(widx_hbm, pidx_hbm, word_hbm, pos_hbm, out_hbm,
            wi_v, pi_v, biota_v, rows_v, tbuf, pos_sh,
            isem, wsem, psem, osem):
    sid = lax.axis_index("s")
    wid = sid * NC + lax.axis_index("c")
    bc = wid % BC
    lg = wid // BC
    l0 = lg * LPW

    @pl.when(sid == 0)
    def _stage_pos():
        pltpu.sync_copy(pos_hbm, pos_sh)

    for g in range(8):
        biota_v[pl.ds(g * 16, 16)] = lax.iota(jnp.int32, 16) + g * 16
    plsc.subcore_barrier()

    def idx_cp(j, b):
        off = pl.multiple_of((l0 + j) * B + bc * CHUNK, CHUNK)
        return (pltpu.make_async_copy(widx_hbm.at[pl.ds(off, CHUNK)],
                                      wi_v.at[b], isem),
                pltpu.make_async_copy(pidx_hbm.at[pl.ds(off, CHUNK)],
                                      pi_v.at[b], isem))

    def word_gather(b):
        return pltpu.make_async_copy(word_hbm.at[wi_v.at[b]],
                                     rows_v.at[b], wsem)

    def pos_add(b):
        return pltpu.async_copy(pos_sh.at[pi_v.at[b]], rows_v.at[b], psem,
                                add=True)

    def out_cp(j, b):
        return pltpu.make_async_copy(
            tbuf.at[b], out_hbm.at[l0 + j, :, pl.ds(bc, 1)], osem)

    def extract(b):
        rows = [biota_v[pl.ds(g * 16, 16)] for g in range(8)]
        src = rows_v.at[b]

        for r in range(8):
            for s in range(8):
                dvec = jnp.full((16,), r * 8 + s, jnp.int32)
                for g in range(8):
                    v = plsc.load_gather(src, [rows[g], dvec])
                    tbuf[b, r, 0, s, pl.ds(g * 16, 16)] = v

    ic0 = idx_cp(0, 0)
    ic0[0].start(); ic0[1].start()
    ic1 = idx_cp(1, 1)
    ic1[0].start(); ic1[1].start()
    ic0[0].wait(); ic0[1].wait()
    word_gather(0).start()

    def body(i, carry):
        for half in range(2):
            j = 2 * i + half
            buf = half
            nbuf = 1 - half

            @pl.when(j + 1 < LPW)
            def _advance():
                ic = idx_cp(j + 1, nbuf)
                ic[0].wait(); ic[1].wait()
                word_gather(nbuf).start()

            @pl.when(j >= 2)
            def _reclaim_tbuf():
                out_cp(j - 2, buf).wait()

            word_gather(buf).wait()
            pos_add(buf).wait()
            extract(buf)
            out_cp(j, buf).start()

            @pl.when(j + 2 < LPW)
            def _next_idx():
                ic2 = idx_cp(j + 2, buf)
                ic2[0].start(); ic2[1].start()
        return carry

    lax.fori_loop(0, LPW // 2, body, 0)
    out_cp(LPW - 2, 0).wait()
    out_cp(LPW - 1, 1).wait()


def kernel(input_data, pos_data, word_table, pos_table):
    widx = input_data.T.reshape(N_IDX).astype(jnp.int32)
    pidx = pos_data.T.reshape(N_IDX).astype(jnp.int32)
    out5 = _lookup(widx, pidx, word_table, pos_table)
    return out5.transpose(2, 4, 0, 1, 3).reshape(B, L, EMB)


# speedup vs baseline: 1.2414x; 1.2414x over previous
import functools

import jax
import jax.numpy as jnp
from jax import lax
from jax.experimental import pallas as pl
from jax.experimental.pallas import tpu as pltpu
from jax.experimental.pallas import tpu_sc as plsc

VOCAB = 1000000
MAX_LEN = 2048
EMB = 64
B, L = 1024, 200
N_IDX = B * L
NC, NS = 2, 16
NW = NC * NS
CHUNK = 128
ROWS_PER_W = N_IDX // NW
CHUNKS_PER_W = ROWS_PER_W // CHUNK
NBUF = 5
N_ITER = CHUNKS_PER_W // NBUF

_mesh = plsc.VectorSubcoreMesh(
    core_axis_name="c", subcore_axis_name="s", num_cores=NC, num_subcores=NS
)


@functools.partial(
    pl.kernel,
    out_type=jax.ShapeDtypeStruct((N_IDX, EMB), jnp.float32),
    mesh=_mesh,
    compiler_params=pltpu.CompilerParams(use_tc_tiling_on_sc=False),
    scratch_types=[
        pltpu.VMEM((ROWS_PER_W,), jnp.int32),
        pltpu.VMEM((ROWS_PER_W,), jnp.int32),
        pltpu.VMEM((NBUF, CHUNK, EMB), jnp.float32),
        pltpu.VMEM_SHARED((MAX_LEN, EMB), jnp.float32),
        pltpu.SemaphoreType.DMA,
        pltpu.SemaphoreType.DMA,
        pltpu.SemaphoreType.DMA,
    ],
)
def _emb_lookup(widx_hbm, pidx_hbm, word_hbm, pos_hbm, out_hbm,
                widx_v, pidx_v, rows_v, pos_sh, wsem, asem, osem):
    sid = lax.axis_index("s")
    wid = sid * NC + lax.axis_index("c")
    out_base = wid * ROWS_PER_W

    icp1 = pltpu.async_copy(widx_hbm.at[pl.ds(out_base, ROWS_PER_W)], widx_v, wsem)
    icp2 = pltpu.async_copy(pidx_hbm.at[pl.ds(out_base, ROWS_PER_W)], pidx_v, wsem)

    @pl.when(sid == 0)
    def _stage_pos_table():
        pltpu.sync_copy(pos_hbm, pos_sh)

    icp1.wait()
    icp2.wait()
    plsc.subcore_barrier()

    def word_cp(j, b):
        return pltpu.async_copy(
            word_hbm.at[widx_v.at[pl.ds(j * CHUNK, CHUNK)]], rows_v.at[b], wsem)

    def pos_cp(j, b):
        return pltpu.async_copy(
            pos_sh.at[pidx_v.at[pl.ds(j * CHUNK, CHUNK)]], rows_v.at[b], asem,
            add=True)

    def out_cp(j, b):
        return pltpu.make_async_copy(
            rows_v.at[b], out_hbm.at[pl.ds(out_base + j * CHUNK, CHUNK)], osem)

    def body(i, carry):
        j0 = i * NBUF

        @pl.when(i > 0)
        def _reclaim_buffers():
            for b in range(NBUF):
                out_cp(j0 - NBUF + b, b).wait()

        wcps = [word_cp(j0 + b, b) for b in range(NBUF)]
        acps = []
        for b in range(NBUF):
            wcps[b].wait()
            acps.append(pos_cp(j0 + b, b))
        for b in range(NBUF):
            acps[b].wait()
            out_cp(j0 + b, b).start()
        return carry

    lax.fori_loop(0, N_ITER, body, 0)
    for b in range(NBUF):
        out_cp((N_ITER - 1) * NBUF + b, b).wait()


def kernel(input_data, pos_data, word_table, pos_table):
    widx = input_data.reshape(N_IDX).astype(jnp.int32)
    pidx = pos_data.reshape(N_IDX).astype(jnp.int32)
    out = _emb_lookup(widx, pidx, word_table, pos_table)
    return out.reshape(B, L, EMB)
